# baseline (device time: 44260 ns/iter reference)
import jax
import jax.numpy as jnp
from jax import lax
from jax.experimental import pallas as pl
from jax.experimental.pallas import tpu as pltpu

B, S, H, Dh, Dr = 2, 256, 16, 64, 32
D = 1024
DC_SH = 64
BS = B * S


def _dot(a, b):
    return jnp.dot(a, b, preferred_element_type=jnp.float32)


def _dot_t(a, b):
    return lax.dot_general(
        a, b, (((1,), (1,)), ((), ())), preferred_element_type=jnp.float32
    )


def kernel(x, Wdkv, Wuk, Wuv, Wq, Wqr, Wkr, Wo):
    def body(
        x_ref, wdkv_ref, wuk_ref, wuv_ref, wq_ref, wqr_ref, wkr_ref, wo_ref,
        out_ref,
        c_buf, c_rem, wuk_rem, wuv_rem, o_buf,
        send_sems, recv_sems,
    ):
        my_x = lax.axis_index("x")
        my_y = lax.axis_index("y")
        nbr = (1 - my_x, my_y)

        barrier_sem = pltpu.get_barrier_semaphore()
        pl.semaphore_signal(
            barrier_sem, inc=1, device_id=nbr,
            device_id_type=pl.DeviceIdType.MESH,
        )
        pl.semaphore_wait(barrier_sem, 1)

        x2 = x_ref[:].reshape(BS, D)
        c_buf[:] = _dot(x2, wdkv_ref[:])

        rdmas = []
        for i, (src, dst) in enumerate(
            [(c_buf, c_rem), (wuk_ref, wuk_rem), (wuv_ref, wuv_rem)]
        ):
            r = pltpu.make_async_remote_copy(
                src_ref=src, dst_ref=dst,
                send_sem=send_sems.at[i], recv_sem=recv_sems.at[i],
                device_id=nbr, device_id_type=pl.DeviceIdType.MESH,
            )
            r.start()
            rdmas.append(r)

        q = _dot(x2, wq_ref[:])
        qr = _dot(x2, wqr_ref[:])
        kr = _dot(x2, wkr_ref[:])
        k_part = _dot(c_buf[:], wuk_ref[:])
        v_part = _dot(c_buf[:], wuv_ref[:])

        for r in rdmas:
            r.wait()

        k = k_part + _dot(c_rem[:], wuk_rem[:])
        v = v_part + _dot(c_rem[:], wuv_rem[:])

        scale = (Dh + Dr) ** -0.5
        for b in range(B):
            kr_b = kr[b * S:(b + 1) * S, :]
            for h in range(H):
                q_bh = q[b * S:(b + 1) * S, h * Dh:(h + 1) * Dh]
                k_bh = k[b * S:(b + 1) * S, h * Dh:(h + 1) * Dh]
                qr_bh = qr[b * S:(b + 1) * S, h * Dr:(h + 1) * Dr]
                s = (_dot_t(q_bh, k_bh) + _dot_t(qr_bh, kr_b)) * scale
                m = jnp.max(s, axis=-1, keepdims=True)
                p = jnp.exp(s - m)
                p = p / jnp.sum(p, axis=-1, keepdims=True)
                v_bh = v[b * S:(b + 1) * S, h * Dh:(h + 1) * Dh]
                o_buf[b * S:(b + 1) * S, h * Dh:(h + 1) * Dh] = _dot(p, v_bh)

        out_ref[:] = _dot(o_buf[:], wo_ref[:]).reshape(B, S, D)

    return pl.pallas_call(
        body,
        out_shape=jax.ShapeDtypeStruct((B, S, D), jnp.float32),
        in_specs=[pl.BlockSpec(memory_space=pltpu.VMEM)] * 8,
        out_specs=pl.BlockSpec(memory_space=pltpu.VMEM),
        scratch_shapes=[
            pltpu.VMEM((BS, DC_SH), jnp.float32),
            pltpu.VMEM((BS, DC_SH), jnp.float32),
            pltpu.VMEM((DC_SH, D), jnp.float32),
            pltpu.VMEM((DC_SH, D), jnp.float32),
            pltpu.VMEM((BS, H * Dh), jnp.float32),
            pltpu.SemaphoreType.DMA((3,)),
            pltpu.SemaphoreType.DMA((3,)),
        ],
        compiler_params=pltpu.CompilerParams(collective_id=0),
    )(x, Wdkv, Wuk, Wuv, Wq, Wqr, Wkr, Wo)
